# baseline (speedup 1.0000x reference)
import jax
import jax.numpy as jnp
from jax.experimental import pallas as pl
from jax.experimental.pallas import tpu as pltpu
from jax.experimental.pallas import tpu_sc as plsc

B = 16384
D = 64

OFF_H, OFF_W, OFF_D, OFF_P = 0, 24, 31, 41
NR = 8400
NRP = 8448


def _vector_mesh():
    return plsc.VectorSubcoreMesh(core_axis_name="core",
                                  subcore_axis_name="subcore")


def _build_block(h_ref, w_ref, d_ref, p_ref, tcat_ref, W_ref, b_ref,
                 g_ref, be_ref, tab_ref, cidx_ref):
    ri = jax.lax.broadcasted_iota(jnp.int32, (64, 64), 0)
    tcat = tcat_ref[...]
    fused = jnp.zeros((64, 64), jnp.float32)
    for f, (lo, hi) in enumerate(((OFF_H, OFF_W), (OFF_W, OFF_D),
                                  (OFF_D, OFF_P), (OFF_P, 46))):
        mask = (ri >= lo) & (ri < hi)
        part = jnp.where(mask, tcat, 0.0)
        fused = fused + jnp.dot(part, W_ref[pl.ds(64 * f, 64), :],
                                preferred_element_type=jnp.float32)

    r = jax.lax.broadcasted_iota(jnp.int32, (8, NRP), 1)
    q350 = r // 350
    q50 = r // 50
    q5 = r // 5
    hr = q350
    wr = q50 - 7 * q350 + OFF_W
    dr = q5 - 10 * q50 + OFF_D
    pr = r - 5 * q5 + OFF_P
    ci = jax.lax.broadcasted_iota(jnp.int32, (64, NRP), 0)
    mh = ((ci == hr[0:1]).astype(jnp.float32)
          + (ci == wr[0:1]).astype(jnp.float32)
          + (ci == dr[0:1]).astype(jnp.float32)
          + (ci == pr[0:1]).astype(jnp.float32))
    x = jax.lax.dot_general(mh, fused, (((0,), (0,)), ((), ())),
                            preferred_element_type=jnp.float32)
    x = x + b_ref[...]
    mu = jnp.mean(x, axis=1, keepdims=True)
    xc = x - mu
    var = jnp.mean(xc * xc, axis=1, keepdims=True)
    y = xc * jax.lax.rsqrt(var + 1e-5) * g_ref[...] + be_ref[...]
    y = jnp.maximum(y, 0.0)
    tab_ref[...] = jnp.concatenate([y, y], axis=1)

    h = h_ref[...]
    cidx_ref[...] = ((h * 7 + w_ref[...]) * 10 + d_ref[...]) * 5 + p_ref[...]


def _build_table(hour2, weekday2, device2, platform2, tcat, W, b, gamma, beta,
                 interpret=False):
    full = lambda shape: pl.BlockSpec(shape, lambda: tuple(0 for _ in shape))
    idx_spec = full((16, B // 16))
    return pl.pallas_call(
        _build_block,
        in_specs=[idx_spec, idx_spec, idx_spec, idx_spec,
                  full((64, 64)), full((256, 64)), full((1, 64)),
                  full((1, 64)), full((1, 64))],
        out_specs=(pl.BlockSpec((NRP, 2 * D), lambda: (0, 0)),
                   pl.BlockSpec((16, B // 16), lambda: (0, 0))),
        out_shape=(jax.ShapeDtypeStruct((NRP, 2 * D), jnp.float32),
                   jax.ShapeDtypeStruct((16, B // 16), jnp.int32)),
        interpret=interpret,
    )(hour2, weekday2, device2, platform2, tcat, W, b.reshape(1, D),
      gamma.reshape(1, D), beta.reshape(1, D))


NC, NS = 2, 16
NW = NC * NS
BPW = B // NW


def _sc_gather(table, cidx):
    @pl.kernel(out_type=jax.ShapeDtypeStruct((B // 2, 2 * D), jnp.float32),
               mesh=_vector_mesh(),
               scratch_types=[pltpu.VMEM((BPW,), jnp.int32),
                              pltpu.VMEM((BPW, 2 * D), jnp.float32),
                              pltpu.VMEM((BPW // 2, 2 * D), jnp.float32),
                              pltpu.SemaphoreType.DMA])
    def k(tab_hbm, i_hbm, o_hbm, idx_v, rows_v, pair_v, sem):
        wid = (jax.lax.axis_index("subcore") * NC
               + jax.lax.axis_index("core"))
        base = wid * BPW
        pltpu.sync_copy(i_hbm.at[pl.ds(base, BPW)], idx_v)
        pltpu.async_copy(tab_hbm.at[idx_v], rows_v, sem).wait()

        @pl.loop(0, BPW // 2)
        def _(j):
            for t in range(4):
                pair_v.at[j, pl.ds(16 * t, 16)][...] = (
                    rows_v.at[2 * j, pl.ds(16 * t, 16)][...])
                pair_v.at[j, pl.ds(D + 16 * t, 16)][...] = (
                    rows_v.at[2 * j + 1, pl.ds(16 * t, 16)][...])

        pltpu.sync_copy(pair_v, o_hbm.at[pl.ds(wid * (BPW // 2), BPW // 2)])

    return k(table, cidx)


def kernel(hour, weekday, device, platform, hour_table, weekday_table,
           device_table, platform_table, W, b, gamma, beta):
    tcat = jnp.zeros((64, D), jnp.float32)
    tcat = jax.lax.dynamic_update_slice(tcat, hour_table, (OFF_H, 0))
    tcat = jax.lax.dynamic_update_slice(tcat, weekday_table, (OFF_W, 0))
    tcat = jax.lax.dynamic_update_slice(tcat, device_table, (OFF_D, 0))
    tcat = jax.lax.dynamic_update_slice(tcat, platform_table, (OFF_P, 0))
    r2 = lambda a: a.reshape(16, B // 16)
    table, cidx = _build_table(r2(hour), r2(weekday), r2(device), r2(platform),
                               tcat, W, b, gamma, beta)
    return _sc_gather(table, cidx.reshape(B)).reshape(B, D)


# speedup vs baseline: 1.2187x; 1.2187x over previous
import jax
import jax.numpy as jnp
from jax.experimental import pallas as pl
from jax.experimental.pallas import tpu as pltpu
from jax.experimental.pallas import tpu_sc as plsc

B = 16384
D = 64

OFF_H, OFF_W, OFF_D, OFF_P = 0, 24, 31, 41
NROWS = 48

NR = 8400
NRP = 8448

B_SC = 4096
B_TC = B - B_SC
BB = 2048
NB = B_TC // BB

NC, NS = 2, 16
NW = NC * NS
BPW = B_SC // NW


def _fused_centered(tcat_ref, W_ref, b_ref):
    ri = jax.lax.broadcasted_iota(jnp.int32, (NROWS, 64), 0)
    tcat48 = tcat_ref[pl.ds(0, NROWS), :]
    fused = jnp.zeros((NROWS, 64), jnp.float32)
    for f, (lo, hi) in enumerate(((OFF_H, OFF_W), (OFF_W, OFF_D),
                                  (OFF_D, OFF_P), (OFF_P, 46))):
        mask = (ri >= lo) & (ri < hi)
        part = jnp.where(mask, tcat48, 0.0)
        fused = fused + jnp.dot(part, W_ref[pl.ds(64 * f, 64), :],
                                preferred_element_type=jnp.float32)
    fused = fused - jnp.mean(fused, axis=1, keepdims=True)
    bc = b_ref[...] - jnp.mean(b_ref[...])
    is_p = (ri >= OFF_P) & (ri < 46)
    fused = jnp.where(is_p, fused + bc, fused)
    hi16 = fused.astype(jnp.bfloat16)
    lo16 = (fused - hi16.astype(jnp.float32)).astype(jnp.bfloat16)
    return hi16, lo16


def _norm_tail(xc, g_ref, be_ref):
    sq = xc * xc
    ones = jnp.full((64, 64), 1.0 / 64.0, jnp.float32)
    var = jnp.dot(sq, ones, preferred_element_type=jnp.float32)
    rs = jax.lax.rsqrt(var + 1e-5)
    y = xc * (rs * g_ref[...]) + be_ref[...]
    return jnp.maximum(y, 0.0)


def _encode_block(h_ref, w_ref, d_ref, p_ref, tcat_ref, W_ref, b_ref,
                  g_ref, be_ref, out_ref):
    hi16, lo16 = _fused_centered(tcat_ref, W_ref, b_ref)
    ci = jax.lax.broadcasted_iota(jnp.int32, (NROWS, BB), 0).astype(jnp.bfloat16)
    one = jnp.ones((), jnp.bfloat16)
    zero = jnp.zeros((), jnp.bfloat16)
    mh = (jnp.where(ci == h_ref[0].astype(jnp.bfloat16), one, zero)
          + jnp.where(ci == w_ref[0].astype(jnp.bfloat16), one, zero)
          + jnp.where(ci == d_ref[0].astype(jnp.bfloat16), one, zero)
          + jnp.where(ci == p_ref[0].astype(jnp.bfloat16), one, zero))
    dn = (((0,), (0,)), ((), ()))
    xc = (jax.lax.dot_general(mh, hi16, dn, preferred_element_type=jnp.float32)
          + jax.lax.dot_general(mh, lo16, dn,
                                preferred_element_type=jnp.float32))
    out_ref[...] = _norm_tail(xc, g_ref, be_ref)


def _encode_tc(hour, weekday, device, platform, tcat, W, b, gamma, beta,
               interpret=False):
    idx3 = lambda a: a[:B_TC].reshape(NB, 1, BB)
    idx_spec = pl.BlockSpec((1, 1, BB), lambda i: (i, 0, 0))
    full = lambda shape: pl.BlockSpec(shape, lambda i: tuple(0 for _ in shape))
    return pl.pallas_call(
        _encode_block,
        grid=(NB,),
        in_specs=[idx_spec, idx_spec, idx_spec, idx_spec,
                  full((64, 64)), full((256, 64)), full((1, 64)),
                  full((1, 64)), full((1, 64))],
        out_specs=pl.BlockSpec((BB, 64), lambda i: (i, 0)),
        out_shape=jax.ShapeDtypeStruct((B_TC, D), jnp.float32),
        interpret=interpret,
    )(idx3(hour), idx3(weekday), idx3(device), idx3(platform),
      tcat, W, b.reshape(1, D), gamma.reshape(1, D), beta.reshape(1, D))


def _build_block(tcat_ref, W_ref, b_ref, g_ref, be_ref, tab_ref):
    hi16, lo16 = _fused_centered(tcat_ref, W_ref, b_ref)
    r = jax.lax.broadcasted_iota(jnp.int32, (8, NRP), 1)
    q350 = r // 350
    q50 = r // 50
    q5 = r // 5
    to16 = lambda a: a[0:1].astype(jnp.bfloat16)
    hr = to16(q350)
    wr = to16(q50 - 7 * q350 + OFF_W)
    dr = to16(q5 - 10 * q50 + OFF_D)
    pr = to16(r - 5 * q5 + OFF_P)
    ci = jax.lax.broadcasted_iota(jnp.int32, (NROWS, NRP), 0).astype(jnp.bfloat16)
    one = jnp.ones((), jnp.bfloat16)
    zero = jnp.zeros((), jnp.bfloat16)
    mh = (jnp.where(ci == hr, one, zero) + jnp.where(ci == wr, one, zero)
          + jnp.where(ci == dr, one, zero) + jnp.where(ci == pr, one, zero))
    dn = (((0,), (0,)), ((), ()))
    xc = (jax.lax.dot_general(mh, hi16, dn, preferred_element_type=jnp.float32)
          + jax.lax.dot_general(mh, lo16, dn,
                                preferred_element_type=jnp.float32))
    y = _norm_tail(xc, g_ref, be_ref)
    tab_ref[:, pl.ds(0, D)] = y
    tab_ref[:, pl.ds(D, D)] = y


def _build_table(tcat, W, b, gamma, beta, interpret=False):
    full = lambda shape: pl.BlockSpec(shape, lambda: tuple(0 for _ in shape))
    return pl.pallas_call(
        _build_block,
        in_specs=[full((64, 64)), full((256, 64)), full((1, 64)),
                  full((1, 64)), full((1, 64))],
        out_specs=pl.BlockSpec((NRP, 2 * D), lambda: (0, 0)),
        out_shape=jax.ShapeDtypeStruct((NRP, 2 * D), jnp.float32),
        interpret=interpret,
    )(tcat, W, b.reshape(1, D), gamma.reshape(1, D), beta.reshape(1, D))


def _vector_mesh():
    return plsc.VectorSubcoreMesh(core_axis_name="core",
                                  subcore_axis_name="subcore")


def _sc_gather(table, hour, weekday, device, platform):
    @pl.kernel(out_type=jax.ShapeDtypeStruct((B_SC // 2, 2 * D), jnp.float32),
               mesh=_vector_mesh(),
               scratch_types=[pltpu.VMEM((BPW,), jnp.int32),
                              pltpu.VMEM((BPW,), jnp.int32),
                              pltpu.VMEM((BPW, 2 * D), jnp.float32),
                              pltpu.VMEM((BPW // 2, 2 * D), jnp.float32),
                              pltpu.SemaphoreType.DMA])
    def k(tab_hbm, h_hbm, w_hbm, d_hbm, p_hbm, o_hbm,
          idx_v, tmp_v, rows_v, pair_v, sem):
        wid = (jax.lax.axis_index("subcore") * NC
               + jax.lax.axis_index("core"))
        base = B_TC + wid * BPW
        pltpu.sync_copy(h_hbm.at[pl.ds(base, BPW)], idx_v)
        pltpu.sync_copy(w_hbm.at[pl.ds(base, BPW)], tmp_v)

        @pl.loop(0, BPW, step=16)
        def _(j):
            s = pl.ds(j, 16)
            idx_v.at[s][...] = idx_v.at[s][...] * 7 + tmp_v.at[s][...]

        pltpu.sync_copy(d_hbm.at[pl.ds(base, BPW)], tmp_v)

        @pl.loop(0, BPW, step=16)
        def _(j):
            s = pl.ds(j, 16)
            idx_v.at[s][...] = idx_v.at[s][...] * 10 + tmp_v.at[s][...]

        pltpu.sync_copy(p_hbm.at[pl.ds(base, BPW)], tmp_v)

        @pl.loop(0, BPW, step=16)
        def _(j):
            s = pl.ds(j, 16)
            idx_v.at[s][...] = idx_v.at[s][...] * 5 + tmp_v.at[s][...]

        pltpu.async_copy(tab_hbm.at[idx_v], rows_v, sem).wait()

        @pl.loop(0, BPW // 2)
        def _(j):
            for t in range(4):
                pair_v.at[j, pl.ds(16 * t, 16)][...] = (
                    rows_v.at[2 * j, pl.ds(16 * t, 16)][...])
                pair_v.at[j, pl.ds(D + 16 * t, 16)][...] = (
                    rows_v.at[2 * j + 1, pl.ds(16 * t, 16)][...])

        pltpu.sync_copy(pair_v, o_hbm.at[pl.ds(wid * (BPW // 2), BPW // 2)])

    return k(table, hour, weekday, device, platform)


def kernel(hour, weekday, device, platform, hour_table, weekday_table,
           device_table, platform_table, W, b, gamma, beta):
    tcat = jnp.zeros((64, D), jnp.float32)
    tcat = jax.lax.dynamic_update_slice(tcat, hour_table, (OFF_H, 0))
    tcat = jax.lax.dynamic_update_slice(tcat, weekday_table, (OFF_W, 0))
    tcat = jax.lax.dynamic_update_slice(tcat, device_table, (OFF_D, 0))
    tcat = jax.lax.dynamic_update_slice(tcat, platform_table, (OFF_P, 0))
    wofs = weekday + OFF_W
    dofs = device + OFF_D
    pofs = platform + OFF_P
    table = _build_table(tcat, W, b, gamma, beta)
    out_sc = _sc_gather(table, hour, weekday, device, platform)
    out_tc = _encode_tc(hour, wofs, dofs, pofs, tcat, W, b, gamma, beta)
    return jnp.concatenate([out_tc, out_sc.reshape(B_SC, D)], axis=0)
